# baseline (speedup 1.0000x reference)
import functools

import jax
import jax.numpy as jnp
from jax import lax
from jax.experimental import pallas as pl
from jax.experimental.pallas import tpu as pltpu
from jax.experimental.pallas import tpu_sc as plsc

_VOCAB = 1000000
_NUM_LABELS = 50
_WORD_DIM = 50
_POS_DIM = 15


def _sc_gather(table, idx, B, D):
    info = plsc.get_sparse_core_info()
    nw = info.num_cores * info.num_subcores
    b_per_w = B // nw
    mesh = plsc.VectorSubcoreMesh(core_axis_name="c", subcore_axis_name="s")

    @functools.partial(
        pl.kernel,
        mesh=mesh,
        out_type=jax.ShapeDtypeStruct((B, D), jnp.float32),
        scratch_types=[
            pltpu.VMEM((b_per_w,), jnp.int32),
            pltpu.VMEM((b_per_w, D), jnp.float32),
            pltpu.SemaphoreType.DMA,
        ],
    )
    def gather_k(table_hbm, idx_hbm, out_hbm, idx_v, rows_v, sem):
        wid = lax.axis_index("s") * info.num_cores + lax.axis_index("c")
        base = wid * b_per_w
        pltpu.sync_copy(idx_hbm.at[pl.ds(base, b_per_w)], idx_v)

        def body(g, carry):
            vec = idx_v[pl.ds(g * 16, 16)]
            for j in range(16):
                r = vec[j]
                pltpu.async_copy(
                    table_hbm.at[pl.ds(r, 1)],
                    rows_v.at[pl.ds(g * 16 + j, 1)],
                    sem,
                )
            return carry

        lax.fori_loop(0, b_per_w // 16, body, 0)
        pltpu.make_async_copy(
            table_hbm.at[pl.ds(0, b_per_w)], rows_v, sem
        ).wait()
        pltpu.sync_copy(rows_v, out_hbm.at[pl.ds(base, b_per_w)])

    return gather_k(table, idx)


def _tc_body(emb_ref, pos_ref, W_ref, ptab_ref, b_ref, out_ref):
    x = emb_ref[...]
    W = W_ref[...]
    Ww = W[:, :_WORD_DIM]
    Wp = W[:, _WORD_DIM:]
    P = lax.dot_general(ptab_ref[...], Wp, (((1,), (1,)), ((), ())),
                        precision=lax.Precision.HIGHEST)
    labels = lax.broadcasted_iota(jnp.int32, (1, _NUM_LABELS), 1)
    onehot = (pos_ref[...] == labels).astype(jnp.float32)
    scores = lax.dot_general(x, Ww, (((1,), (1,)), ((), ())),
                             precision=lax.Precision.HIGHEST)
    scores = scores + lax.dot_general(onehot, P, (((1,), (0,)), ((), ())),
                                      precision=lax.Precision.HIGHEST)
    out_ref[...] = scores + b_ref[...]


def kernel(word_ids, prev_pos, word_table, pos_table, W, b):
    B = word_ids.shape[0]
    emb = _sc_gather(word_table, word_ids.astype(jnp.int32), B, _WORD_DIM)
    if True:
        pos_emb = jnp.take(pos_table, prev_pos, axis=0)
        x = jnp.concatenate([emb, pos_emb], axis=1)
        return x @ W.T + b

    blk = 2048
    grid = (B // blk,)
    scores = pl.pallas_call(
        _tc_body,
        grid=grid,
        in_specs=[
            pl.BlockSpec((blk, _WORD_DIM), lambda i: (i, 0)),
            pl.BlockSpec((blk, 1), lambda i: (i, 0)),
            pl.BlockSpec((_NUM_LABELS, _WORD_DIM + _POS_DIM), lambda i: (0, 0)),
            pl.BlockSpec((_NUM_LABELS, _POS_DIM), lambda i: (0, 0)),
            pl.BlockSpec((1, _NUM_LABELS), lambda i: (0, 0)),
        ],
        out_specs=pl.BlockSpec((blk, _NUM_LABELS), lambda i: (i, 0)),
        out_shape=jax.ShapeDtypeStruct((B, _NUM_LABELS), jnp.float32),
    )(emb, prev_pos.astype(jnp.int32).reshape(B, 1), W, pos_table,
      b.reshape(1, _NUM_LABELS))
    return scores


# speedup vs baseline: 1.0987x; 1.0987x over previous
import functools

import jax
import jax.numpy as jnp
from jax import lax
from jax.experimental import pallas as pl
from jax.experimental.pallas import tpu as pltpu
from jax.experimental.pallas import tpu_sc as plsc

_VOCAB = 1000000
_NUM_LABELS = 50
_WORD_DIM = 50
_POS_DIM = 15


def _sc_gather(table, idx, B, D):
    info = plsc.get_sparse_core_info()
    nw = info.num_cores * info.num_subcores
    b_per_w = B // nw
    mesh = plsc.VectorSubcoreMesh(core_axis_name="c", subcore_axis_name="s")

    @functools.partial(
        pl.kernel,
        mesh=mesh,
        out_type=jax.ShapeDtypeStruct((B, D), jnp.float32),
        compiler_params=pltpu.CompilerParams(use_tc_tiling_on_sc=True),
        scratch_types=[
            pltpu.VMEM((b_per_w,), jnp.int32),
            pltpu.VMEM((b_per_w, D), jnp.float32),
            pltpu.SemaphoreType.DMA,
        ],
    )
    def gather_k(table_hbm, idx_hbm, out_hbm, idx_v, rows_v, sem):
        wid = lax.axis_index("s") * info.num_cores + lax.axis_index("c")
        base = wid * b_per_w
        pltpu.sync_copy(idx_hbm.at[pl.ds(base, b_per_w)], idx_v)

        def body(g, carry):
            vec = idx_v[pl.ds(g * 16, 16)]
            for j in range(16):
                r = vec[j]
                pltpu.async_copy(
                    table_hbm.at[pl.ds(r, 1)],
                    rows_v.at[pl.ds(g * 16 + j, 1)],
                    sem,
                )
            return carry

        lax.fori_loop(0, b_per_w // 16, body, 0)
        pltpu.make_async_copy(
            table_hbm.at[pl.ds(0, b_per_w)], rows_v, sem
        ).wait()
        pltpu.sync_copy(rows_v, out_hbm.at[pl.ds(base, b_per_w)])

    return gather_k(table, idx)


def _tc_body(emb_ref, pos_ref, W_ref, ptab_ref, b_ref, out_ref):
    x = emb_ref[...]
    W = W_ref[...]
    Ww = W[:, :_WORD_DIM]
    Wp = W[:, _WORD_DIM:]
    P = lax.dot_general(ptab_ref[...], Wp, (((1,), (1,)), ((), ())),
                        precision=lax.Precision.HIGHEST)
    labels = lax.broadcasted_iota(jnp.int32, (1, _NUM_LABELS), 1)
    onehot = (pos_ref[...] == labels).astype(jnp.float32)
    scores = lax.dot_general(x, Ww, (((1,), (1,)), ((), ())),
                             precision=lax.Precision.HIGHEST)
    scores = scores + lax.dot_general(onehot, P, (((1,), (0,)), ((), ())),
                                      precision=lax.Precision.HIGHEST)
    out_ref[...] = scores + b_ref[...]


def kernel(word_ids, prev_pos, word_table, pos_table, W, b):
    B = word_ids.shape[0]
    emb = _sc_gather(word_table, word_ids.astype(jnp.int32), B, _WORD_DIM)
    blk = 2048
    grid = (B // blk,)
    scores = pl.pallas_call(
        _tc_body,
        grid=grid,
        in_specs=[
            pl.BlockSpec((blk, _WORD_DIM), lambda i: (i, 0)),
            pl.BlockSpec((blk, 1), lambda i: (i, 0)),
            pl.BlockSpec((_NUM_LABELS, _WORD_DIM + _POS_DIM), lambda i: (0, 0)),
            pl.BlockSpec((_NUM_LABELS, _POS_DIM), lambda i: (0, 0)),
            pl.BlockSpec((1, _NUM_LABELS), lambda i: (0, 0)),
        ],
        out_specs=pl.BlockSpec((blk, _NUM_LABELS), lambda i: (i, 0)),
        out_shape=jax.ShapeDtypeStruct((B, _NUM_LABELS), jnp.float32),
    )(emb, prev_pos.astype(jnp.int32).reshape(B, 1), W, pos_table,
      b.reshape(1, _NUM_LABELS))
    return scores


# speedup vs baseline: 1.4852x; 1.3518x over previous
import functools

import jax
import jax.numpy as jnp
from jax import lax
from jax.experimental import pallas as pl
from jax.experimental.pallas import tpu as pltpu
from jax.experimental.pallas import tpu_sc as plsc

_NUM_LABELS = 50
_WORD_DIM = 50
_POS_DIM = 15


def _proj_body(tabt_ref, W_ref, ptab_ref, b_ref, out_ref, pout_ref):
    xt = tabt_ref[...]
    W = W_ref[...]
    Ww = W[:, :_WORD_DIM]
    scores = lax.dot_general(
        xt.astype(jnp.bfloat16), Ww.astype(jnp.bfloat16),
        (((0,), (1,)), ((), ())), preferred_element_type=jnp.float32)
    out_ref[...] = scores + b_ref[...]

    @pl.when(pl.program_id(0) == 0)
    def _():
        Wp = W[:, _WORD_DIM:]
        pout_ref[...] = lax.dot_general(ptab_ref[...], Wp,
                                        (((1,), (1,)), ((), ())),
                                        precision=lax.Precision.HIGHEST)


def _project(table_t, pos_table, W, b, V):
    blk = 16384
    grid = (pl.cdiv(V, blk),)
    return pl.pallas_call(
        _proj_body,
        grid=grid,
        in_specs=[
            pl.BlockSpec((_WORD_DIM, blk), lambda i: (0, i)),
            pl.BlockSpec((_NUM_LABELS, _WORD_DIM + _POS_DIM), lambda i: (0, 0)),
            pl.BlockSpec((_NUM_LABELS, _POS_DIM), lambda i: (0, 0)),
            pl.BlockSpec((1, _NUM_LABELS), lambda i: (0, 0)),
        ],
        out_specs=[
            pl.BlockSpec((blk, _NUM_LABELS), lambda i: (i, 0)),
            pl.BlockSpec((_NUM_LABELS, _NUM_LABELS), lambda i: (0, 0)),
        ],
        out_shape=[
            jax.ShapeDtypeStruct((V, _NUM_LABELS), jnp.float32),
            jax.ShapeDtypeStruct((_NUM_LABELS, _NUM_LABELS), jnp.float32),
        ],
    )(table_t, W, pos_table, b.reshape(1, _NUM_LABELS))


def _sc_gather(table, idx, B, D):
    info = plsc.get_sparse_core_info()
    nw = info.num_cores * info.num_subcores
    b_per_w = B // nw
    mesh = plsc.VectorSubcoreMesh(core_axis_name="c", subcore_axis_name="s")

    @functools.partial(
        pl.kernel,
        mesh=mesh,
        out_type=jax.ShapeDtypeStruct((B, D), jnp.float32),
        compiler_params=pltpu.CompilerParams(use_tc_tiling_on_sc=True),
        scratch_types=[
            pltpu.VMEM((b_per_w,), jnp.int32),
            pltpu.VMEM((b_per_w, D), jnp.float32),
            pltpu.SemaphoreType.DMA,
        ],
    )
    def gather_k(table_hbm, idx_hbm, out_hbm, idx_v, rows_v, sem):
        wid = lax.axis_index("s") * info.num_cores + lax.axis_index("c")
        base = wid * b_per_w
        pltpu.sync_copy(idx_hbm.at[pl.ds(base, b_per_w)], idx_v)

        def body(g, carry):
            vec = idx_v[pl.ds(g * 16, 16)]
            for j in range(16):
                r = vec[j]
                pltpu.async_copy(
                    table_hbm.at[pl.ds(r, 1)],
                    rows_v.at[pl.ds(g * 16 + j, 1)],
                    sem,
                )
            return carry

        lax.fori_loop(0, b_per_w // 16, body, 0)
        pltpu.make_async_copy(
            table_hbm.at[pl.ds(0, b_per_w)], rows_v, sem
        ).wait()
        pltpu.sync_copy(rows_v, out_hbm.at[pl.ds(base, b_per_w)])

    return gather_k(table, idx)


def _epi_body(x_ref, pos_ref, ppos_ref, out_ref):
    labels = lax.broadcasted_iota(jnp.int32, (1, _NUM_LABELS), 1)
    onehot = (pos_ref[...] == labels).astype(jnp.float32)
    out_ref[...] = x_ref[...] + lax.dot_general(
        onehot, ppos_ref[...], (((1,), (0,)), ((), ())),
        precision=lax.Precision.HIGHEST)


def kernel(word_ids, prev_pos, word_table, pos_table, W, b):
    B = word_ids.shape[0]
    V = word_table.shape[0]
    p_tab, p_pos = _project(word_table.T, pos_table, W, b, V)
    word_scores = _sc_gather(p_tab, word_ids.astype(jnp.int32), B, _NUM_LABELS)

    blk = 2048
    grid = (B // blk,)
    scores = pl.pallas_call(
        _epi_body,
        grid=grid,
        in_specs=[
            pl.BlockSpec((blk, _NUM_LABELS), lambda i: (i, 0)),
            pl.BlockSpec((blk, 1), lambda i: (i, 0)),
            pl.BlockSpec((_NUM_LABELS, _NUM_LABELS), lambda i: (0, 0)),
        ],
        out_specs=pl.BlockSpec((blk, _NUM_LABELS), lambda i: (i, 0)),
        out_shape=jax.ShapeDtypeStruct((B, _NUM_LABELS), jnp.float32),
    )(word_scores, prev_pos.astype(jnp.int32).reshape(B, 1), p_pos)
    return scores


# speedup vs baseline: 1.5194x; 1.0230x over previous
import functools

import jax
import jax.numpy as jnp
from jax import lax
from jax.experimental import pallas as pl
from jax.experimental.pallas import tpu as pltpu
from jax.experimental.pallas import tpu_sc as plsc

_NUM_LABELS = 50
_WORD_DIM = 50
_POS_DIM = 15


def _proj_body(tabt_ref, W_ref, ptab_ref, b_ref, out_ref, pout_ref):
    xt = tabt_ref[...]
    W = W_ref[...]
    Ww = W[:, :_WORD_DIM]
    scores = lax.dot_general(
        xt.astype(jnp.bfloat16), Ww.astype(jnp.bfloat16),
        (((0,), (1,)), ((), ())), preferred_element_type=jnp.float32)
    out_ref[...] = scores + b_ref[...]

    @pl.when(pl.program_id(0) == 0)
    def _():
        Wp = W[:, _WORD_DIM:]
        pout_ref[...] = lax.dot_general(ptab_ref[...], Wp,
                                        (((1,), (1,)), ((), ())),
                                        precision=lax.Precision.HIGHEST)


def _project(table_t, pos_table, W, b, V):
    blk = 32768
    grid = (pl.cdiv(V, blk),)
    return pl.pallas_call(
        _proj_body,
        grid=grid,
        in_specs=[
            pl.BlockSpec((_WORD_DIM, blk), lambda i: (0, i)),
            pl.BlockSpec((_NUM_LABELS, _WORD_DIM + _POS_DIM), lambda i: (0, 0)),
            pl.BlockSpec((_NUM_LABELS, _POS_DIM), lambda i: (0, 0)),
            pl.BlockSpec((1, _NUM_LABELS), lambda i: (0, 0)),
        ],
        out_specs=[
            pl.BlockSpec((blk, _NUM_LABELS), lambda i: (i, 0)),
            pl.BlockSpec((_NUM_LABELS, _NUM_LABELS), lambda i: (0, 0)),
        ],
        out_shape=[
            jax.ShapeDtypeStruct((V, _NUM_LABELS), jnp.float32),
            jax.ShapeDtypeStruct((_NUM_LABELS, _NUM_LABELS), jnp.float32),
        ],
    )(table_t, W, pos_table, b.reshape(1, _NUM_LABELS))


def _sc_gather(table, idx, B, D):
    info = plsc.get_sparse_core_info()
    nw = info.num_cores * info.num_subcores
    b_per_w = B // nw
    mesh = plsc.VectorSubcoreMesh(core_axis_name="c", subcore_axis_name="s")

    @functools.partial(
        pl.kernel,
        mesh=mesh,
        out_type=jax.ShapeDtypeStruct((B, D), jnp.float32),
        compiler_params=pltpu.CompilerParams(use_tc_tiling_on_sc=True),
        scratch_types=[
            pltpu.VMEM((b_per_w,), jnp.int32),
            pltpu.VMEM((b_per_w, D), jnp.float32),
            pltpu.SemaphoreType.DMA,
        ],
    )
    def gather_k(table_hbm, idx_hbm, out_hbm, idx_v, rows_v, sem):
        wid = lax.axis_index("s") * info.num_cores + lax.axis_index("c")
        base = wid * b_per_w
        pltpu.sync_copy(idx_hbm.at[pl.ds(base, b_per_w)], idx_v)

        def body(g, carry):
            vec = idx_v[pl.ds(g * 16, 16)]
            for j in range(16):
                r = vec[j]
                pltpu.async_copy(
                    table_hbm.at[pl.ds(r, 1)],
                    rows_v.at[pl.ds(g * 16 + j, 1)],
                    sem,
                )
            return carry

        lax.fori_loop(0, b_per_w // 16, body, 0)
        pltpu.make_async_copy(
            table_hbm.at[pl.ds(0, b_per_w)], rows_v, sem
        ).wait()
        pltpu.sync_copy(rows_v, out_hbm.at[pl.ds(base, b_per_w)])

    return gather_k(table, idx)


def _epi_body(x_ref, pos_ref, ppos_ref, out_ref):
    labels = lax.broadcasted_iota(jnp.int32, (1, _NUM_LABELS), 1)
    onehot = (pos_ref[...] == labels).astype(jnp.float32)
    out_ref[...] = x_ref[...] + lax.dot_general(
        onehot, ppos_ref[...], (((1,), (0,)), ((), ())),
        precision=lax.Precision.HIGHEST)


def kernel(word_ids, prev_pos, word_table, pos_table, W, b):
    B = word_ids.shape[0]
    V = word_table.shape[0]
    p_tab, p_pos = _project(word_table.T, pos_table, W, b, V)
    word_scores = _sc_gather(p_tab, word_ids.astype(jnp.int32), B, _NUM_LABELS)

    blk = 2048
    grid = (B // blk,)
    scores = pl.pallas_call(
        _epi_body,
        grid=grid,
        in_specs=[
            pl.BlockSpec((blk, _NUM_LABELS), lambda i: (i, 0)),
            pl.BlockSpec((blk, 1), lambda i: (i, 0)),
            pl.BlockSpec((_NUM_LABELS, _NUM_LABELS), lambda i: (0, 0)),
        ],
        out_specs=pl.BlockSpec((blk, _NUM_LABELS), lambda i: (i, 0)),
        out_shape=jax.ShapeDtypeStruct((B, _NUM_LABELS), jnp.float32),
    )(word_scores, prev_pos.astype(jnp.int32).reshape(B, 1), p_pos)
    return scores
